# baseline (speedup 1.0000x reference)
import functools

import jax
import jax.numpy as jnp
from jax.experimental import pallas as pl


def _mm_body(x_ref, w_ref, b_ref, o_ref, *, in_tanh):
    x = x_ref[...]
    if in_tanh:
        x = jnp.tanh(x)
    acc = jnp.dot(x, w_ref[...], preferred_element_type=jnp.float32)
    if b_ref is not None:
        acc = acc + b_ref[...]
    o_ref[...] = acc


def _mm_body_nob(x_ref, w_ref, o_ref, *, in_tanh):
    _mm_body(x_ref, w_ref, None, o_ref, in_tanh=in_tanh)


def _matmul(x, w, b=None, in_tanh=False, block_rows=2048):
    R, K = x.shape
    F = w.shape[1]
    cap = max(256, (1 << 19) // max(K, F) // 8 * 8)
    block_rows = min(block_rows, cap)
    Rp = ((R + block_rows - 1) // block_rows) * block_rows
    if Rp != R:
        x = jnp.pad(x, ((0, Rp - R), (0, 0)))
    grid = (Rp // block_rows,)
    in_specs = [
        pl.BlockSpec((block_rows, K), lambda i: (i, 0)),
        pl.BlockSpec((K, F), lambda i: (0, 0)),
    ]
    args = [x, w]
    if b is not None:
        in_specs.append(pl.BlockSpec((1, F), lambda i: (0, 0)))
        args.append(b.reshape(1, F))
        body = functools.partial(_mm_body, in_tanh=in_tanh)
    else:
        body = functools.partial(_mm_body_nob, in_tanh=in_tanh)
    out = pl.pallas_call(
        body,
        grid=grid,
        in_specs=in_specs,
        out_specs=pl.BlockSpec((block_rows, F), lambda i: (i, 0)),
        out_shape=jax.ShapeDtypeStruct((Rp, F), jnp.float32),
    )(*args)
    return out[:R] if Rp != R else out


def _graph_norm(x, p):
    n = x.shape[0]
    mean = jnp.mean(x, axis=0)
    out = x - p["mean_scale"] * mean
    var = jnp.mean(out * out, axis=0)
    return p["gamma"] * out / jnp.sqrt(var + 1e-5) + p["beta"]


def _edge_conv(y, src, dst, p, n):
    xi = y[dst]
    xj = y[src]
    h = jnp.concatenate([xi, xj - xi], axis=-1)
    G = _matmul(h, p["l1"]["W"].T, p["l1"]["b"], block_rows=4096)
    M = _matmul(G, p["l2"]["W"].T, in_tanh=True, block_rows=4096)
    m = jax.ops.segment_max(M, dst, num_segments=n)
    return jnp.where(jnp.isfinite(m), m + p["l2"]["b"], 0.0)


def _graph_conv(y, src, dst, p, n):
    agg = jax.ops.segment_sum(y[src], dst, num_segments=n)
    return _matmul(agg, p["rel"]["W"].T, p["rel"]["b"]) + _matmul(
        y, p["root"]["W"].T)


def kernel(x, edge_index, batch, params):
    n = x.shape[0]
    src, dst = edge_index[0], edge_index[1]
    y0 = _graph_norm(x, params["norm"])
    y1 = jnp.tanh(_edge_conv(y0, src, dst, params["conv000"], n))
    y2 = jnp.tanh(_edge_conv(y1, src, dst, params["conv00"], n))
    y3 = jnp.tanh(_edge_conv(y2, src, dst, params["conv01"], n))
    y4 = jnp.tanh(_edge_conv(y3, src, dst, params["conv02"], n))
    y5 = jnp.tanh(_graph_conv(y4, src, dst, params["conv1"], n))
    y6 = jnp.tanh(_graph_conv(y5, src, dst, params["conv2"], n))
    y7 = jnp.tanh(_graph_conv(y6, src, dst, params["conv3"], n))
    y8 = jnp.tanh(_graph_conv(y7, src, dst, params["conv4"], n)) + y7
    y9 = jnp.tanh(_graph_conv(y8, src, dst, params["conv5"], n)) + y6
    y10 = jnp.tanh(_graph_conv(y9, src, dst, params["conv6"], n)) + y5
    y11 = jnp.tanh(_graph_conv(y10, src, dst, params["conv7"], n)) + y3
    y12 = jnp.tanh(_graph_conv(y11, src, dst, params["conv8"], n)) + y2
    y13 = jnp.tanh(_graph_conv(y12, src, dst, params["conv9"], n)) + y1
    return _graph_conv(y13, src, dst, params["conv10"], n)


# speedup vs baseline: 1.8010x; 1.8010x over previous
import functools

import jax
import jax.numpy as jnp
from jax import lax
from jax.experimental import pallas as pl
from jax.experimental.pallas import tpu as pltpu
from jax.experimental.pallas import tpu_sc as plsc

_N = 10000
_NPT = 320
_REG = 328
_N2 = 32 * _NPT
_B = 128


def _mm_body(x_ref, w_ref, b_ref, o_ref, *, in_tanh):
    x = x_ref[...]
    if in_tanh:
        x = jnp.tanh(x)
    acc = jnp.dot(x, w_ref[...], preferred_element_type=jnp.float32)
    if b_ref is not None:
        acc = acc + b_ref[...]
    o_ref[...] = acc


def _mm_body_nob(x_ref, w_ref, o_ref, *, in_tanh):
    _mm_body(x_ref, w_ref, None, o_ref, in_tanh=in_tanh)


def _matmul(x, w, b=None, in_tanh=False, block_rows=2048):
    R, K = x.shape
    F = w.shape[1]
    cap = max(256, (1 << 19) // max(K, F) // 8 * 8)
    block_rows = min(block_rows, cap)
    Rp = ((R + block_rows - 1) // block_rows) * block_rows
    if Rp != R:
        x = jnp.pad(x, ((0, Rp - R), (0, 0)))
    grid = (Rp // block_rows,)
    in_specs = [
        pl.BlockSpec((block_rows, K), lambda i: (i, 0)),
        pl.BlockSpec((K, F), lambda i: (0, 0)),
    ]
    args = [x, w]
    if b is not None:
        in_specs.append(pl.BlockSpec((1, F), lambda i: (0, 0)))
        args.append(b.reshape(1, F))
        body = functools.partial(_mm_body, in_tanh=in_tanh)
    else:
        body = functools.partial(_mm_body_nob, in_tanh=in_tanh)
    out = pl.pallas_call(
        body,
        grid=grid,
        in_specs=in_specs,
        out_specs=pl.BlockSpec((block_rows, F), lambda i: (i, 0)),
        out_shape=jax.ShapeDtypeStruct((Rp, F), jnp.float32),
    )(*args)
    return out[:R] if Rp != R else out


@functools.lru_cache(maxsize=None)
def _sc_seg_sum_call(K, C):
    mesh = plsc.VectorSubcoreMesh(core_axis_name="c", subcore_axis_name="s")

    def body(zc_h, src_h, dst_h, bnd_h, out_h,
             srcv, dstv, idxv, dstlv, bndv, stage, zbuf, shared, sem):
        c = lax.axis_index("c")
        s = lax.axis_index("s")
        w = c * 16 + s
        lo = w * _NPT
        pltpu.sync_copy(bnd_h, bndv)
        bv = bndv[pl.ds(w, 16)]
        start = bv[0]
        end = bv[1]
        start8 = jnp.bitwise_and(start, -8)
        nb = (end - start8 + (_B - 1)) // _B

        def zrow(i, carry):
            for kk in range(C // 16):
                zbuf[i, pl.ds(kk * 16, 16)] = jnp.zeros((16,), jnp.float32)
            return carry

        lax.fori_loop(0, _REG, zrow, 0)
        for cc in range(K):
            pltpu.sync_copy(zbuf, shared.at[pl.ds(s * _REG, _REG)])

            def batch(b, carry):
                base = pl.multiple_of(start8 + b * _B, 8)
                pltpu.sync_copy(src_h.at[pl.ds(base, _B)], srcv)
                pltpu.sync_copy(dst_h.at[pl.ds(base, _B)], dstv)
                for i in range(_B // 16):
                    sl = pl.ds(i * 16, 16)
                    idxv[sl] = srcv[sl] + cc * _N2
                    dl = dstv[sl] - lo
                    ok = (dl >= 0) & (dl < _NPT)
                    dstlv[sl] = jnp.where(ok, dl, _NPT) + s * _REG
                pltpu.async_copy(zc_h.at[idxv], stage, sem).wait()
                pltpu.sync_copy(stage, shared.at[dstlv], add=True)
                return carry

            lax.fori_loop(0, nb, batch, 0)
            pltpu.sync_copy(shared.at[pl.ds(s * _REG, _NPT)],
                            out_h.at[pl.ds(cc * _N2 + w * _NPT, _NPT)])

    return pl.kernel(
        body,
        out_type=jax.ShapeDtypeStruct((K * _N2, C), jnp.float32),
        mesh=mesh,
        scratch_types=[
            pltpu.VMEM((_B,), jnp.int32),
            pltpu.VMEM((_B,), jnp.int32),
            pltpu.VMEM((_B,), jnp.int32),
            pltpu.VMEM((_B,), jnp.int32),
            pltpu.VMEM((48,), jnp.int32),
            pltpu.VMEM((_B, C), jnp.float32),
            pltpu.VMEM((_REG, C), jnp.float32),
            pltpu.VMEM_SHARED((16 * _REG, C), jnp.float32),
            pltpu.SemaphoreType.DMA,
        ],
    )


def _sc_segment_sum(y, src_s, dst_s, bounds):
    n, fin = y.shape
    C = 128
    K = -(-fin // C)
    fp = K * C
    y2 = jnp.pad(y, ((0, _N2 - n), (0, fp - fin)))
    if K > 1:
        zc = y2.reshape(_N2, K, C).transpose(1, 0, 2).reshape(K * _N2, C)
    else:
        zc = y2
    out = _sc_seg_sum_call(K, C)(zc, src_s, dst_s, bounds)
    if K > 1:
        out = out.reshape(K, _N2, C).transpose(1, 0, 2).reshape(_N2, fp)
    return out[:n, :fin]


def _graph_norm(x, batch, p):
    ones = jnp.ones((x.shape[0], 1), x.dtype)
    cnt = jax.ops.segment_sum(ones, batch, num_segments=1)
    mean = jax.ops.segment_sum(x, batch, num_segments=1) / cnt
    out = x - p["mean_scale"] * mean[batch]
    var = jax.ops.segment_sum(out * out, batch, num_segments=1) / cnt
    return p["gamma"] * out / jnp.sqrt(var + 1e-5)[batch] + p["beta"]


def _edge_conv(y, src, dst, p, n):
    xi = y[dst]
    xj = y[src]
    h = jnp.concatenate([xi, xj - xi], axis=-1)
    G = _matmul(h, p["l1"]["W"].T, p["l1"]["b"], block_rows=4096)
    M = _matmul(G, p["l2"]["W"].T, in_tanh=True, block_rows=4096)
    m = jax.ops.segment_max(M, dst, num_segments=n)
    return jnp.where(jnp.isfinite(m), m + p["l2"]["b"], 0.0)


def _graph_conv(y, src_s, dst_s, bounds, p):
    agg = _sc_segment_sum(y, src_s, dst_s, bounds)
    return _matmul(agg, p["rel"]["W"].T, p["rel"]["b"]) + _matmul(
        y, p["root"]["W"].T)


def kernel(x, edge_index, batch, params):
    n = x.shape[0]
    src, dst = edge_index[0], edge_index[1]
    dst_s, src_s = lax.sort_key_val(dst, src)
    bounds = jnp.searchsorted(dst_s, jnp.arange(33, dtype=jnp.int32) * _NPT,
                              method="scan_unrolled").astype(jnp.int32)
    bounds = jnp.pad(bounds, (0, 15))
    dst_s = jnp.pad(dst_s, (0, 2 * _B), constant_values=2 ** 20)
    src_s = jnp.pad(src_s, (0, 2 * _B))
    y0 = _graph_norm(x, batch, params["norm"])
    y1 = jnp.tanh(_edge_conv(y0, src, dst, params["conv000"], n))
    y2 = jnp.tanh(_edge_conv(y1, src, dst, params["conv00"], n))
    y3 = jnp.tanh(_edge_conv(y2, src, dst, params["conv01"], n))
    y4 = jnp.tanh(_edge_conv(y3, src, dst, params["conv02"], n))
    y5 = jnp.tanh(_graph_conv(y4, src_s, dst_s, bounds, params["conv1"]))
    y6 = jnp.tanh(_graph_conv(y5, src_s, dst_s, bounds, params["conv2"]))
    y7 = jnp.tanh(_graph_conv(y6, src_s, dst_s, bounds, params["conv3"]))
    y8 = jnp.tanh(_graph_conv(y7, src_s, dst_s, bounds, params["conv4"])) + y7
    y9 = jnp.tanh(_graph_conv(y8, src_s, dst_s, bounds, params["conv5"])) + y6
    y10 = jnp.tanh(_graph_conv(y9, src_s, dst_s, bounds, params["conv6"])) + y5
    y11 = jnp.tanh(_graph_conv(y10, src_s, dst_s, bounds, params["conv7"])) + y3
    y12 = jnp.tanh(_graph_conv(y11, src_s, dst_s, bounds, params["conv8"])) + y2
    y13 = jnp.tanh(_graph_conv(y12, src_s, dst_s, bounds, params["conv9"])) + y1
    return _graph_conv(y13, src_s, dst_s, bounds, params["conv10"])
